# Initial kernel scaffold; baseline (speedup 1.0000x reference)
import jax
import jax.numpy as jnp
from jax.experimental import pallas as pl


def kernel(x_fine, edge_index_fine, edge_index_coarse, cluster_ids, W_embed, b_embed, Wf0, bf0, Vf0, Uf0, Wf1, bf1, Vf1, Uf1, Wc0, bc0, Vc0, Uc0, Wc1, bc1, Vc1, Uc1, W_dec, b_dec):
    raise NotImplementedError("write your pallas kernel here")


# speedup vs baseline: 5.4621x; 5.4621x over previous
import functools

import jax
import jax.numpy as jnp
from jax import lax
from jax.experimental import pallas as pl
from jax.experimental.pallas import tpu as pltpu
from jax.experimental.pallas import tpu_sc as plsc

_NF = 10000
_NC = 2500
_EF = 320000
_EC = 40000
_D = 128
_H = 128
_BOND = 16

_NCORES = 2
_NSUB = 16
_NW = _NCORES * _NSUB

_NPF = 10240
_NPC = 2560

_CH = 128
_KF = 80
_EFP = _NW * _KF * _CH
_KC = 16
_ECP = _NW * _KC * _CH
_AF = 10368
_AC = 2560

_RCH = 40
_KR = 8

_F32 = jnp.float32
_HIGH = lax.Precision.HIGHEST


def _dot(a, b):
    return jnp.dot(a, b, precision=_HIGH, preferred_element_type=_F32)


def _seg_sum16_body(table_ref, src_ref, dst_ref, zero_ref, out_ref,
                    srcv, dstv, rows, stage, acc, sem, *, K, RPT):
    cid = lax.axis_index("c")
    sid = lax.axis_index("s")
    wid = cid * _NSUB + sid
    base = wid * K
    pltpu.sync_copy(src_ref.at[pl.ds(base, K)], srcv)
    pltpu.sync_copy(dst_ref.at[pl.ds(base, K)], dstv)
    pltpu.sync_copy(zero_ref.at[pl.ds(sid * RPT, RPT)], stage)
    pltpu.sync_copy(stage, acc.at[pl.ds(sid * RPT, RPT)])
    plsc.subcore_barrier()

    def chunk(k, carry):
        pltpu.async_copy(table_ref.at[srcv.at[k]], rows, sem).wait()
        pltpu.sync_copy(rows, acc.at[dstv.at[k]], add=True)
        return carry

    lax.fori_loop(0, K, chunk, 0)
    plsc.subcore_barrier()
    pltpu.sync_copy(acc.at[pl.ds(sid * RPT, RPT)], stage)
    pltpu.sync_copy(stage, out_ref.at[cid, pl.ds(sid * RPT, RPT)])


def _seg_sum16(table, src2d, dst2d, *, K, NACC):
    RPT = NACC // _NSUB
    mesh = plsc.VectorSubcoreMesh(core_axis_name="c", subcore_axis_name="s")
    zero = jnp.zeros((NACC, 16), _F32)
    kern = pl.kernel(
        functools.partial(_seg_sum16_body, K=K, RPT=RPT),
        out_type=jax.ShapeDtypeStruct((_NCORES, NACC, 16), _F32),
        mesh=mesh,
        compiler_params=pltpu.CompilerParams(use_tc_tiling_on_sc=False),
        scratch_types=[
            pltpu.VMEM((K, _CH), jnp.int32),
            pltpu.VMEM((K, _CH), jnp.int32),
            pltpu.VMEM((_CH, 16), _F32),
            pltpu.VMEM((RPT, 16), _F32),
            pltpu.VMEM_SHARED((NACC, 16), _F32),
            pltpu.SemaphoreType.DMA,
        ],
    )
    return kern(table, src2d, dst2d, zero)


def _restrict_body(h_ref, cid_ref, zero128_ref, zero16_ref, ones_ref,
                   sums_ref, cnts_ref,
                   cidv, hbuf, onesb, stage, stage16, acc, acc16, sem):
    cid = lax.axis_index("c")
    sid = lax.axis_index("s")
    wid = cid * _NSUB + sid
    RPT = _NPC // _NSUB
    pltpu.sync_copy(cid_ref.at[pl.ds(wid * _KR, _KR)], cidv)
    pltpu.sync_copy(ones_ref, onesb)
    pltpu.sync_copy(zero128_ref.at[pl.ds(sid * RPT, RPT)], stage)
    pltpu.sync_copy(stage, acc.at[pl.ds(sid * RPT, RPT)])
    pltpu.sync_copy(zero16_ref.at[pl.ds(sid * RPT, RPT)], stage16)
    pltpu.sync_copy(stage16, acc16.at[pl.ds(sid * RPT, RPT)])
    plsc.subcore_barrier()

    def chunk(k, carry):
        pltpu.sync_copy(h_ref.at[pl.ds(wid * (_KR * _RCH) + k * _RCH, _RCH)],
                        hbuf)
        pltpu.sync_copy(hbuf, acc.at[cidv.at[k]], add=True)
        pltpu.sync_copy(onesb, acc16.at[cidv.at[k]], add=True)
        return carry

    lax.fori_loop(0, _KR, chunk, 0)
    plsc.subcore_barrier()
    pltpu.sync_copy(acc.at[pl.ds(sid * RPT, RPT)], stage)
    pltpu.sync_copy(stage, sums_ref.at[cid, pl.ds(sid * RPT, RPT)])
    pltpu.sync_copy(acc16.at[pl.ds(sid * RPT, RPT)], stage16)
    pltpu.sync_copy(stage16, cnts_ref.at[cid, pl.ds(sid * RPT, RPT)])


def _restrict(h, cid2d):
    RPT = _NPC // _NSUB
    mesh = plsc.VectorSubcoreMesh(core_axis_name="c", subcore_axis_name="s")
    zero128 = jnp.zeros((_NPC, 128), _F32)
    zero16 = jnp.zeros((_NPC, 16), _F32)
    ones = jnp.ones((_RCH, 16), _F32)
    kern = pl.kernel(
        _restrict_body,
        out_type=(jax.ShapeDtypeStruct((_NCORES, _NPC, 128), _F32),
                  jax.ShapeDtypeStruct((_NCORES, _NPC, 16), _F32)),
        mesh=mesh,
        compiler_params=pltpu.CompilerParams(use_tc_tiling_on_sc=False),
        scratch_types=[
            pltpu.VMEM((_KR, _RCH), jnp.int32),
            pltpu.VMEM((_RCH, 128), _F32),
            pltpu.VMEM((_RCH, 16), _F32),
            pltpu.VMEM((RPT, 128), _F32),
            pltpu.VMEM((RPT, 16), _F32),
            pltpu.VMEM_SHARED((_NPC, 128), _F32),
            pltpu.VMEM_SHARED((_NPC, 16), _F32),
            pltpu.SemaphoreType.DMA,
        ],
    )
    return kern(h, cid2d, zero128, zero16, ones)


def _prolong_body(hc_ref, cid_ref, out_ref, cidv, hbuf, sem):
    cid = lax.axis_index("c")
    sid = lax.axis_index("s")
    wid = cid * _NSUB + sid
    pltpu.sync_copy(cid_ref.at[pl.ds(wid * _KR, _KR)], cidv)

    def chunk(k, carry):
        pltpu.async_copy(hc_ref.at[cidv.at[k]], hbuf, sem).wait()
        pltpu.sync_copy(hbuf,
                        out_ref.at[pl.ds(wid * (_KR * _RCH) + k * _RCH, _RCH)])
        return carry

    lax.fori_loop(0, _KR, chunk, 0)


def _prolong(hc, cid2d):
    mesh = plsc.VectorSubcoreMesh(core_axis_name="c", subcore_axis_name="s")
    kern = pl.kernel(
        _prolong_body,
        out_type=jax.ShapeDtypeStruct((_NPF, 128), _F32),
        mesh=mesh,
        compiler_params=pltpu.CompilerParams(use_tc_tiling_on_sc=False),
        scratch_types=[
            pltpu.VMEM((_KR, _RCH), jnp.int32),
            pltpu.VMEM((_RCH, 128), _F32),
            pltpu.SemaphoreType.DMA,
        ],
    )
    return kern(hc, cid2d)


def _embed_body(x_ref, wet_ref, be_ref, vt_ref, h_ref, g_ref):
    h = jnp.maximum(_dot(x_ref[...], wet_ref[...]) + be_ref[...], 0.0)
    h_ref[...] = h
    g_ref[...] = _dot(h, vt_ref[...])


def _tc_embed(x, Wet, be, Vt, BM):
    R = x.shape[0]
    grid = R // BM
    return pl.pallas_call(
        _embed_body,
        grid=(grid,),
        in_specs=[
            pl.BlockSpec((BM, _D), lambda i: (i, 0)),
            pl.BlockSpec((_D, _H), lambda i: (0, 0)),
            pl.BlockSpec((1, _H), lambda i: (0, 0)),
            pl.BlockSpec((_H, _BOND), lambda i: (0, 0)),
        ],
        out_specs=[
            pl.BlockSpec((BM, _H), lambda i: (i, 0)),
            pl.BlockSpec((BM, _BOND), lambda i: (i, 0)),
        ],
        out_shape=[
            jax.ShapeDtypeStruct((R, _H), _F32),
            jax.ShapeDtypeStruct((R, _BOND), _F32),
        ],
    )(x, Wet, be, Vt)


def _layer_body(h_ref, a0_ref, a1_ref, wt_ref, b_ref, ut_ref, vt_ref,
                hn_ref, g_ref):
    agg = a0_ref[...] + a1_ref[...]
    hn = _dot(h_ref[...], wt_ref[...]) + b_ref[...] + _dot(agg, ut_ref[...])
    hn = jnp.maximum(hn, 0.0)
    hn_ref[...] = hn
    if g_ref is not None:
        g_ref[...] = _dot(hn, vt_ref[...])


def _tc_layer(h, a0, a1, Wt, b, Ut, Vt, BM, with_g):
    R = h.shape[0]
    grid = R // BM
    out_shape = [jax.ShapeDtypeStruct((R, _H), _F32)]
    out_specs = [pl.BlockSpec((BM, _H), lambda i: (i, 0))]
    if with_g:
        out_shape.append(jax.ShapeDtypeStruct((R, _BOND), _F32))
        out_specs.append(pl.BlockSpec((BM, _BOND), lambda i: (i, 0)))
        body = _layer_body
    else:
        def body(h_ref, a0_ref, a1_ref, wt_ref, b_ref, ut_ref, vt_ref, hn_ref):
            _layer_body(h_ref, a0_ref, a1_ref, wt_ref, b_ref, ut_ref, vt_ref,
                        hn_ref, None)
    args = [h, a0, a1, Wt, b, Ut]
    in_specs = [
        pl.BlockSpec((BM, _H), lambda i: (i, 0)),
        pl.BlockSpec((BM, _BOND), lambda i: (i, 0)),
        pl.BlockSpec((BM, _BOND), lambda i: (i, 0)),
        pl.BlockSpec((_H, _H), lambda i: (0, 0)),
        pl.BlockSpec((1, _H), lambda i: (0, 0)),
        pl.BlockSpec((_BOND, _H), lambda i: (0, 0)),
    ]
    if with_g:
        args.append(Vt)
        in_specs.append(pl.BlockSpec((_H, _BOND), lambda i: (0, 0)))
        full_body = body
    else:
        def full_body(h_ref, a0_ref, a1_ref, wt_ref, b_ref, ut_ref, hn_ref):
            body(h_ref, a0_ref, a1_ref, wt_ref, b_ref, ut_ref, None, hn_ref)
    res = pl.pallas_call(
        full_body,
        grid=(grid,),
        in_specs=in_specs,
        out_specs=out_specs,
        out_shape=out_shape,
    )(*args)
    return res if with_g else (res[0], None)


def _mean_body(s0_ref, s1_ref, c0_ref, c1_ref, vt_ref, hc_ref, g_ref):
    cnt = c0_ref[...][:, 0:1] + c1_ref[...][:, 0:1]
    hc = (s0_ref[...] + s1_ref[...]) / jnp.maximum(cnt, 1.0)
    hc_ref[...] = hc
    g_ref[...] = _dot(hc, vt_ref[...])


def _tc_mean(s0, s1, c0, c1, Vt):
    R = s0.shape[0]
    return pl.pallas_call(
        _mean_body,
        grid=(1,),
        in_specs=[
            pl.BlockSpec((R, _H), lambda i: (0, 0)),
            pl.BlockSpec((R, _H), lambda i: (0, 0)),
            pl.BlockSpec((R, 16), lambda i: (0, 0)),
            pl.BlockSpec((R, 16), lambda i: (0, 0)),
            pl.BlockSpec((_H, _BOND), lambda i: (0, 0)),
        ],
        out_specs=[
            pl.BlockSpec((R, _H), lambda i: (0, 0)),
            pl.BlockSpec((R, _BOND), lambda i: (0, 0)),
        ],
        out_shape=[
            jax.ShapeDtypeStruct((R, _H), _F32),
            jax.ShapeDtypeStruct((R, _BOND), _F32),
        ],
    )(s0, s1, c0, c1, Vt)


def _decode_body(h_ref, p_ref, wt_ref, b_ref, o_ref):
    o_ref[...] = _dot(h_ref[...] + p_ref[...], wt_ref[...]) + b_ref[...]


def _tc_decode(h, ph, Wdt, bd, BM):
    R = h.shape[0]
    grid = R // BM
    return pl.pallas_call(
        _decode_body,
        grid=(grid,),
        in_specs=[
            pl.BlockSpec((BM, _H), lambda i: (i, 0)),
            pl.BlockSpec((BM, _H), lambda i: (i, 0)),
            pl.BlockSpec((_H, _D), lambda i: (0, 0)),
            pl.BlockSpec((1, _D), lambda i: (0, 0)),
        ],
        out_specs=pl.BlockSpec((BM, _D), lambda i: (i, 0)),
        out_shape=jax.ShapeDtypeStruct((R, _D), _F32),
    )(h, ph, Wdt, bd)


def _pad_edges(src, dst, E, EP, dummy):
    src_p = jnp.concatenate([src, jnp.zeros((EP - E,), jnp.int32)])
    dst_p = jnp.concatenate([dst, jnp.full((EP - E,), dummy, jnp.int32)])
    return src_p.reshape(-1, _CH), dst_p.reshape(-1, _CH)


@jax.jit
def kernel(x_fine, edge_index_fine, edge_index_coarse, cluster_ids,
           W_embed, b_embed,
           Wf0, bf0, Vf0, Uf0, Wf1, bf1, Vf1, Uf1,
           Wc0, bc0, Vc0, Uc0, Wc1, bc1, Vc1, Uc1, W_dec, b_dec):
    x = jnp.zeros((_NPF, _D), _F32).at[:_NF].set(x_fine)
    srcf2d, dstf2d = _pad_edges(edge_index_fine[0], edge_index_fine[1],
                                _EF, _EFP, _NPF)
    srcc2d, dstc2d = _pad_edges(edge_index_coarse[0], edge_index_coarse[1],
                                _EC, _ECP, _NPC)
    cidp = jnp.concatenate([cluster_ids,
                            jnp.full((_NPF - _NF,), _NC, jnp.int32)])
    cid2d = cidp.reshape(_NW * _KR, _RCH)

    Wet = W_embed.T
    be = b_embed.reshape(1, _H)
    Wts = dict(f0=Wf0.T, f1=Wf1.T, c0=Wc0.T, c1=Wc1.T)
    Uts = dict(f0=Uf0.T, f1=Uf1.T, c0=Uc0.T, c1=Uc1.T)
    Vts = dict(f0=Vf0.T, f1=Vf1.T, c0=Vc0.T, c1=Vc1.T)
    bs = dict(f0=bf0.reshape(1, _H), f1=bf1.reshape(1, _H),
              c0=bc0.reshape(1, _H), c1=bc1.reshape(1, _H))
    Wdt = W_dec.T
    bd = b_dec.reshape(1, _D)

    BMF = 1280

    h0, g0 = _tc_embed(x, Wet, be, Vts["f0"], BMF)

    agg0 = _seg_sum16(g0, srcf2d, dstf2d, K=_KF, NACC=_AF)
    h1, g1 = _tc_layer(h0, agg0[0, :_NPF], agg0[1, :_NPF],
                       Wts["f0"], bs["f0"], Uts["f0"], Vts["f1"], BMF, True)

    agg1 = _seg_sum16(g1, srcf2d, dstf2d, K=_KF, NACC=_AF)
    h2, _ = _tc_layer(h1, agg1[0, :_NPF], agg1[1, :_NPF],
                      Wts["f1"], bs["f1"], Uts["f1"], None, BMF, False)

    sums, cnts = _restrict(h2, cid2d)
    hc0, gc0 = _tc_mean(sums[0], sums[1], cnts[0], cnts[1], Vts["c0"])

    aggc0 = _seg_sum16(gc0, srcc2d, dstc2d, K=_KC, NACC=_AC)
    hc1, gc1 = _tc_layer(hc0, aggc0[0, :_NPC], aggc0[1, :_NPC],
                         Wts["c0"], bs["c0"], Uts["c0"], Vts["c1"], _NPC, True)

    aggc1 = _seg_sum16(gc1, srcc2d, dstc2d, K=_KC, NACC=_AC)
    hc2, _ = _tc_layer(hc1, aggc1[0, :_NPC], aggc1[1, :_NPC],
                       Wts["c1"], bs["c1"], Uts["c1"], None, _NPC, False)

    ph = _prolong(hc2, cid2d)
    out = _tc_decode(h2, ph, Wdt, bd, BMF)
    return out[:_NF]


# speedup vs baseline: 12.0006x; 2.1971x over previous
import functools

import jax
import jax.numpy as jnp
from jax import lax
from jax.experimental import pallas as pl
from jax.experimental.pallas import tpu as pltpu
from jax.experimental.pallas import tpu_sc as plsc

_NF = 10000
_NC = 2500
_EF = 320000
_EC = 40000
_D = 128
_H = 128
_BOND = 16

_NCORES = 2
_NSUB = 16
_NW = _NCORES * _NSUB

_NPF = 10240
_NPC = 2560

_CH = 128
_KF = 80
_EFP = _NW * _KF * _CH
_KC = 16
_ECP = _NW * _KC * _CH
_AF = 10368
_AC = 2560

_RCH = 40
_KR = 8

_F32 = jnp.float32
_HIGH = lax.Precision.HIGHEST


def _dot(a, b):
    return jnp.dot(a, b, precision=_HIGH, preferred_element_type=_F32)


_NBUF = 4


def _seg_sum16_body(table_ref, src_ref, dst_ref, zero_ref, out_ref,
                    srcv, dstv, rows0, rows1, rows2, rows3, stage, acc,
                    gsem, ssem, *, K, RPT):
    cid = lax.axis_index("c")
    sid = lax.axis_index("s")
    wid = cid * _NSUB + sid
    base = wid * K
    rows = [rows0, rows1, rows2, rows3]
    pltpu.sync_copy(src_ref.at[pl.ds(base, K)], srcv)
    pltpu.sync_copy(dst_ref.at[pl.ds(base, K)], dstv)
    pltpu.sync_copy(zero_ref.at[pl.ds(sid * RPT, RPT)], stage)
    pltpu.sync_copy(stage, acc.at[pl.ds(sid * RPT, RPT)])
    plsc.subcore_barrier()

    def group(g, carry):
        k0 = g * _NBUF
        gd = [pltpu.async_copy(table_ref.at[srcv.at[k0 + b]], rows[b], gsem)
              for b in range(_NBUF)]
        sd = []
        for b in range(_NBUF):
            gd[b].wait()
            sd.append(pltpu.async_copy(rows[b], acc.at[dstv.at[k0 + b]],
                                       ssem, add=True))
        for b in range(_NBUF):
            sd[b].wait()
        return carry

    lax.fori_loop(0, K // _NBUF, group, 0)
    plsc.subcore_barrier()
    pltpu.sync_copy(acc.at[pl.ds(sid * RPT, RPT)], stage)
    pltpu.sync_copy(stage, out_ref.at[cid, pl.ds(sid * RPT, RPT)])


def _seg_sum16(table, src2d, dst2d, *, K, NACC):
    RPT = NACC // _NSUB
    mesh = plsc.VectorSubcoreMesh(core_axis_name="c", subcore_axis_name="s")
    zero = jnp.zeros((NACC, 16), _F32)
    kern = pl.kernel(
        functools.partial(_seg_sum16_body, K=K, RPT=RPT),
        out_type=jax.ShapeDtypeStruct((_NCORES, NACC, 16), _F32),
        mesh=mesh,
        compiler_params=pltpu.CompilerParams(use_tc_tiling_on_sc=False),
        scratch_types=[
            pltpu.VMEM((K, _CH), jnp.int32),
            pltpu.VMEM((K, _CH), jnp.int32),
            pltpu.VMEM((_CH, 16), _F32),
            pltpu.VMEM((_CH, 16), _F32),
            pltpu.VMEM((_CH, 16), _F32),
            pltpu.VMEM((_CH, 16), _F32),
            pltpu.VMEM((RPT, 16), _F32),
            pltpu.VMEM_SHARED((NACC, 16), _F32),
            pltpu.SemaphoreType.DMA,
            pltpu.SemaphoreType.DMA,
        ],
    )
    return kern(table, src2d, dst2d, zero)


def _restrict_body(h_ref, cid_ref, zero128_ref, zero16_ref, ones_ref,
                   sums_ref, cnts_ref,
                   cidv, hbuf0, hbuf1, hbuf2, hbuf3, onesb, stage, stage16,
                   acc, acc16, gsem, ssem):
    cid = lax.axis_index("c")
    sid = lax.axis_index("s")
    wid = cid * _NSUB + sid
    RPT = _NPC // _NSUB
    hbuf = [hbuf0, hbuf1, hbuf2, hbuf3]
    pltpu.sync_copy(cid_ref.at[pl.ds(wid * _KR, _KR)], cidv)
    pltpu.sync_copy(ones_ref, onesb)
    pltpu.sync_copy(zero128_ref.at[pl.ds(sid * RPT, RPT)], stage)
    pltpu.sync_copy(stage, acc.at[pl.ds(sid * RPT, RPT)])
    pltpu.sync_copy(zero16_ref.at[pl.ds(sid * RPT, RPT)], stage16)
    pltpu.sync_copy(stage16, acc16.at[pl.ds(sid * RPT, RPT)])
    plsc.subcore_barrier()

    def group(g, carry):
        k0 = g * _NBUF
        gd = [pltpu.async_copy(
                  h_ref.at[pl.ds(wid * (_KR * _RCH) + (k0 + b) * _RCH, _RCH)],
                  hbuf[b], gsem)
              for b in range(_NBUF)]
        sd = []
        for b in range(_NBUF):
            gd[b].wait()
            sd.append(pltpu.async_copy(hbuf[b], acc.at[cidv.at[k0 + b]],
                                       ssem, add=True))
            sd.append(pltpu.async_copy(onesb, acc16.at[cidv.at[k0 + b]],
                                       ssem, add=True))
        for d in sd:
            d.wait()
        return carry

    lax.fori_loop(0, _KR // _NBUF, group, 0)
    plsc.subcore_barrier()
    pltpu.sync_copy(acc.at[pl.ds(sid * RPT, RPT)], stage)
    pltpu.sync_copy(stage, sums_ref.at[cid, pl.ds(sid * RPT, RPT)])
    pltpu.sync_copy(acc16.at[pl.ds(sid * RPT, RPT)], stage16)
    pltpu.sync_copy(stage16, cnts_ref.at[cid, pl.ds(sid * RPT, RPT)])


def _restrict(h, cid2d):
    RPT = _NPC // _NSUB
    mesh = plsc.VectorSubcoreMesh(core_axis_name="c", subcore_axis_name="s")
    zero128 = jnp.zeros((_NPC, 128), _F32)
    zero16 = jnp.zeros((_NPC, 16), _F32)
    ones = jnp.ones((_RCH, 16), _F32)
    kern = pl.kernel(
        _restrict_body,
        out_type=(jax.ShapeDtypeStruct((_NCORES, _NPC, 128), _F32),
                  jax.ShapeDtypeStruct((_NCORES, _NPC, 16), _F32)),
        mesh=mesh,
        compiler_params=pltpu.CompilerParams(use_tc_tiling_on_sc=False),
        scratch_types=[
            pltpu.VMEM((_KR, _RCH), jnp.int32),
            pltpu.VMEM((_RCH, 128), _F32),
            pltpu.VMEM((_RCH, 128), _F32),
            pltpu.VMEM((_RCH, 128), _F32),
            pltpu.VMEM((_RCH, 128), _F32),
            pltpu.VMEM((_RCH, 16), _F32),
            pltpu.VMEM((RPT, 128), _F32),
            pltpu.VMEM((RPT, 16), _F32),
            pltpu.VMEM_SHARED((_NPC, 128), _F32),
            pltpu.VMEM_SHARED((_NPC, 16), _F32),
            pltpu.SemaphoreType.DMA,
            pltpu.SemaphoreType.DMA,
        ],
    )
    return kern(h, cid2d, zero128, zero16, ones)


def _prolong_body(hc_ref, cid_ref, out_ref, cidv, hbuf0, hbuf1, hbuf2, hbuf3,
                  gsem, ssem):
    cid = lax.axis_index("c")
    sid = lax.axis_index("s")
    wid = cid * _NSUB + sid
    hbuf = [hbuf0, hbuf1, hbuf2, hbuf3]
    pltpu.sync_copy(cid_ref.at[pl.ds(wid * _KR, _KR)], cidv)

    def group(g, carry):
        k0 = g * _NBUF
        gd = [pltpu.async_copy(hc_ref.at[cidv.at[k0 + b]], hbuf[b], gsem)
              for b in range(_NBUF)]
        sd = []
        for b in range(_NBUF):
            gd[b].wait()
            sd.append(pltpu.async_copy(
                hbuf[b],
                out_ref.at[pl.ds(wid * (_KR * _RCH) + (k0 + b) * _RCH, _RCH)],
                ssem))
        for d in sd:
            d.wait()
        return carry

    lax.fori_loop(0, _KR // _NBUF, group, 0)


def _prolong(hc, cid2d):
    mesh = plsc.VectorSubcoreMesh(core_axis_name="c", subcore_axis_name="s")
    kern = pl.kernel(
        _prolong_body,
        out_type=jax.ShapeDtypeStruct((_NPF, 128), _F32),
        mesh=mesh,
        compiler_params=pltpu.CompilerParams(use_tc_tiling_on_sc=False),
        scratch_types=[
            pltpu.VMEM((_KR, _RCH), jnp.int32),
            pltpu.VMEM((_RCH, 128), _F32),
            pltpu.VMEM((_RCH, 128), _F32),
            pltpu.VMEM((_RCH, 128), _F32),
            pltpu.VMEM((_RCH, 128), _F32),
            pltpu.SemaphoreType.DMA,
            pltpu.SemaphoreType.DMA,
        ],
    )
    return kern(hc, cid2d)


def _embed_body(x_ref, wet_ref, be_ref, vt_ref, h_ref, g_ref):
    h = jnp.maximum(_dot(x_ref[...], wet_ref[...]) + be_ref[...], 0.0)
    h_ref[...] = h
    g_ref[...] = _dot(h, vt_ref[...])


def _tc_embed(x, Wet, be, Vt, BM):
    R = x.shape[0]
    grid = R // BM
    return pl.pallas_call(
        _embed_body,
        grid=(grid,),
        in_specs=[
            pl.BlockSpec((BM, _D), lambda i: (i, 0)),
            pl.BlockSpec((_D, _H), lambda i: (0, 0)),
            pl.BlockSpec((1, _H), lambda i: (0, 0)),
            pl.BlockSpec((_H, _BOND), lambda i: (0, 0)),
        ],
        out_specs=[
            pl.BlockSpec((BM, _H), lambda i: (i, 0)),
            pl.BlockSpec((BM, _BOND), lambda i: (i, 0)),
        ],
        out_shape=[
            jax.ShapeDtypeStruct((R, _H), _F32),
            jax.ShapeDtypeStruct((R, _BOND), _F32),
        ],
    )(x, Wet, be, Vt)


def _layer_body(h_ref, a0_ref, a1_ref, wt_ref, b_ref, ut_ref, vt_ref,
                hn_ref, g_ref):
    agg = a0_ref[...] + a1_ref[...]
    hn = _dot(h_ref[...], wt_ref[...]) + b_ref[...] + _dot(agg, ut_ref[...])
    hn = jnp.maximum(hn, 0.0)
    hn_ref[...] = hn
    if g_ref is not None:
        g_ref[...] = _dot(hn, vt_ref[...])


def _tc_layer(h, a0, a1, Wt, b, Ut, Vt, BM, with_g):
    R = h.shape[0]
    grid = R // BM
    out_shape = [jax.ShapeDtypeStruct((R, _H), _F32)]
    out_specs = [pl.BlockSpec((BM, _H), lambda i: (i, 0))]
    if with_g:
        out_shape.append(jax.ShapeDtypeStruct((R, _BOND), _F32))
        out_specs.append(pl.BlockSpec((BM, _BOND), lambda i: (i, 0)))
        body = _layer_body
    else:
        def body(h_ref, a0_ref, a1_ref, wt_ref, b_ref, ut_ref, vt_ref, hn_ref):
            _layer_body(h_ref, a0_ref, a1_ref, wt_ref, b_ref, ut_ref, vt_ref,
                        hn_ref, None)
    args = [h, a0, a1, Wt, b, Ut]
    in_specs = [
        pl.BlockSpec((BM, _H), lambda i: (i, 0)),
        pl.BlockSpec((BM, _BOND), lambda i: (i, 0)),
        pl.BlockSpec((BM, _BOND), lambda i: (i, 0)),
        pl.BlockSpec((_H, _H), lambda i: (0, 0)),
        pl.BlockSpec((1, _H), lambda i: (0, 0)),
        pl.BlockSpec((_BOND, _H), lambda i: (0, 0)),
    ]
    if with_g:
        args.append(Vt)
        in_specs.append(pl.BlockSpec((_H, _BOND), lambda i: (0, 0)))
        full_body = body
    else:
        def full_body(h_ref, a0_ref, a1_ref, wt_ref, b_ref, ut_ref, hn_ref):
            body(h_ref, a0_ref, a1_ref, wt_ref, b_ref, ut_ref, None, hn_ref)
    res = pl.pallas_call(
        full_body,
        grid=(grid,),
        in_specs=in_specs,
        out_specs=out_specs,
        out_shape=out_shape,
    )(*args)
    return res if with_g else (res[0], None)


def _mean_body(s0_ref, s1_ref, c0_ref, c1_ref, vt_ref, hc_ref, g_ref):
    cnt = c0_ref[...][:, 0:1] + c1_ref[...][:, 0:1]
    hc = (s0_ref[...] + s1_ref[...]) / jnp.maximum(cnt, 1.0)
    hc_ref[...] = hc
    g_ref[...] = _dot(hc, vt_ref[...])


def _tc_mean(s0, s1, c0, c1, Vt):
    R = s0.shape[0]
    return pl.pallas_call(
        _mean_body,
        grid=(1,),
        in_specs=[
            pl.BlockSpec((R, _H), lambda i: (0, 0)),
            pl.BlockSpec((R, _H), lambda i: (0, 0)),
            pl.BlockSpec((R, 16), lambda i: (0, 0)),
            pl.BlockSpec((R, 16), lambda i: (0, 0)),
            pl.BlockSpec((_H, _BOND), lambda i: (0, 0)),
        ],
        out_specs=[
            pl.BlockSpec((R, _H), lambda i: (0, 0)),
            pl.BlockSpec((R, _BOND), lambda i: (0, 0)),
        ],
        out_shape=[
            jax.ShapeDtypeStruct((R, _H), _F32),
            jax.ShapeDtypeStruct((R, _BOND), _F32),
        ],
    )(s0, s1, c0, c1, Vt)


def _decode_body(h_ref, p_ref, wt_ref, b_ref, o_ref):
    o_ref[...] = _dot(h_ref[...] + p_ref[...], wt_ref[...]) + b_ref[...]


def _tc_decode(h, ph, Wdt, bd, BM):
    R = h.shape[0]
    grid = R // BM
    return pl.pallas_call(
        _decode_body,
        grid=(grid,),
        in_specs=[
            pl.BlockSpec((BM, _H), lambda i: (i, 0)),
            pl.BlockSpec((BM, _H), lambda i: (i, 0)),
            pl.BlockSpec((_H, _D), lambda i: (0, 0)),
            pl.BlockSpec((1, _D), lambda i: (0, 0)),
        ],
        out_specs=pl.BlockSpec((BM, _D), lambda i: (i, 0)),
        out_shape=jax.ShapeDtypeStruct((R, _D), _F32),
    )(h, ph, Wdt, bd)


def _pad_edges(src, dst, E, EP, nsrc, dummy, ndummy):
    pad = jnp.arange(EP - E, dtype=jnp.int32)
    src_p = jnp.concatenate([src, pad % nsrc])
    dst_p = jnp.concatenate([dst, dummy + pad % ndummy])
    return src_p.reshape(-1, _CH), dst_p.reshape(-1, _CH)


@jax.jit
def kernel(x_fine, edge_index_fine, edge_index_coarse, cluster_ids,
           W_embed, b_embed,
           Wf0, bf0, Vf0, Uf0, Wf1, bf1, Vf1, Uf1,
           Wc0, bc0, Vc0, Uc0, Wc1, bc1, Vc1, Uc1, W_dec, b_dec):
    x = jnp.zeros((_NPF, _D), _F32).at[:_NF].set(x_fine)
    srcf2d, dstf2d = _pad_edges(edge_index_fine[0], edge_index_fine[1],
                                _EF, _EFP, _NF, _NPF, _AF - _NPF)
    srcc2d, dstc2d = _pad_edges(edge_index_coarse[0], edge_index_coarse[1],
                                _EC, _ECP, _NC, _NC, _NPC - _NC)
    padc = jnp.arange(_NPF - _NF, dtype=jnp.int32)
    cidp = jnp.concatenate([cluster_ids, _NC + padc % (_NPC - _NC)])
    cid2d = cidp.reshape(_NW * _KR, _RCH)

    Wet = W_embed.T
    be = b_embed.reshape(1, _H)
    Wts = dict(f0=Wf0.T, f1=Wf1.T, c0=Wc0.T, c1=Wc1.T)
    Uts = dict(f0=Uf0.T, f1=Uf1.T, c0=Uc0.T, c1=Uc1.T)
    Vts = dict(f0=Vf0.T, f1=Vf1.T, c0=Vc0.T, c1=Vc1.T)
    bs = dict(f0=bf0.reshape(1, _H), f1=bf1.reshape(1, _H),
              c0=bc0.reshape(1, _H), c1=bc1.reshape(1, _H))
    Wdt = W_dec.T
    bd = b_dec.reshape(1, _D)

    BMF = 1280

    h0, g0 = _tc_embed(x, Wet, be, Vts["f0"], BMF)

    agg0 = _seg_sum16(g0, srcf2d, dstf2d, K=_KF, NACC=_AF)
    h1, g1 = _tc_layer(h0, agg0[0, :_NPF], agg0[1, :_NPF],
                       Wts["f0"], bs["f0"], Uts["f0"], Vts["f1"], BMF, True)

    agg1 = _seg_sum16(g1, srcf2d, dstf2d, K=_KF, NACC=_AF)
    h2, _ = _tc_layer(h1, agg1[0, :_NPF], agg1[1, :_NPF],
                      Wts["f1"], bs["f1"], Uts["f1"], None, BMF, False)

    sums, cnts = _restrict(h2, cid2d)
    hc0, gc0 = _tc_mean(sums[0], sums[1], cnts[0], cnts[1], Vts["c0"])

    aggc0 = _seg_sum16(gc0, srcc2d, dstc2d, K=_KC, NACC=_AC)
    hc1, gc1 = _tc_layer(hc0, aggc0[0, :_NPC], aggc0[1, :_NPC],
                         Wts["c0"], bs["c0"], Uts["c0"], Vts["c1"], _NPC, True)

    aggc1 = _seg_sum16(gc1, srcc2d, dstc2d, K=_KC, NACC=_AC)
    hc2, _ = _tc_layer(hc1, aggc1[0, :_NPC], aggc1[1, :_NPC],
                       Wts["c1"], bs["c1"], Uts["c1"], None, _NPC, False)

    ph = _prolong(hc2, cid2d)
    out = _tc_decode(h2, ph, Wdt, bd, BMF)
    return out[:_NF]


# speedup vs baseline: 15.9160x; 1.3263x over previous
import functools

import jax
import jax.numpy as jnp
from jax import lax
from jax.experimental import pallas as pl
from jax.experimental.pallas import tpu as pltpu
from jax.experimental.pallas import tpu_sc as plsc

_NF = 10000
_NC = 2500
_EF = 320000
_EC = 40000
_D = 128
_H = 128
_BOND = 16

_NCORES = 2
_NSUB = 16
_NW = _NCORES * _NSUB

_NPF = 10240
_NPC = 2560

_CH = 128
_KF = 80
_EFP = _NW * _KF * _CH
_KC = 16
_ECP = _NW * _KC * _CH
_AF = 10368
_AC = 2560

_RCH = 40
_KR = 8

_F32 = jnp.float32


def _dot(a, b):
    return jnp.dot(a, b, preferred_element_type=_F32)


def _seg_sum16_body(table_ref, src_ref, dst_ref, zero_ref, out_ref,
                    srcv, dstv, rows, stage, acc, gsem, ssem, *, K, RPT, NBUF):
    cid = lax.axis_index("c")
    sid = lax.axis_index("s")
    wid = cid * _NSUB + sid
    base = wid * K
    pltpu.sync_copy(src_ref.at[pl.ds(base, K)], srcv)
    pltpu.sync_copy(dst_ref.at[pl.ds(base, K)], dstv)
    pltpu.sync_copy(zero_ref.at[pl.ds(sid * RPT, RPT)], stage)
    pltpu.sync_copy(stage, acc.at[pl.ds(sid * RPT, RPT)])
    plsc.subcore_barrier()

    def group(g, carry):
        k0 = g * NBUF
        gd = [pltpu.async_copy(table_ref.at[srcv.at[k0 + b]], rows[b], gsem)
              for b in range(NBUF)]
        sd = []
        for b in range(NBUF):
            gd[b].wait()
            sd.append(pltpu.async_copy(rows[b], acc.at[dstv.at[k0 + b]],
                                       ssem, add=True))
        for b in range(NBUF):
            sd[b].wait()
        return carry

    lax.fori_loop(0, K // NBUF, group, 0)
    plsc.subcore_barrier()
    pltpu.sync_copy(acc.at[pl.ds(sid * RPT, RPT)], stage)
    pltpu.sync_copy(stage, out_ref.at[cid, pl.ds(sid * RPT, RPT)])


def _seg_sum16(table, src2d, dst2d, *, K, NACC, NBUF=8):
    RPT = NACC // _NSUB
    mesh = plsc.VectorSubcoreMesh(core_axis_name="c", subcore_axis_name="s")
    zero = jnp.zeros((NACC, 16), _F32)
    kern = pl.kernel(
        functools.partial(_seg_sum16_body, K=K, RPT=RPT, NBUF=NBUF),
        out_type=jax.ShapeDtypeStruct((_NCORES, NACC, 16), _F32),
        mesh=mesh,
        compiler_params=pltpu.CompilerParams(use_tc_tiling_on_sc=False),
        scratch_types=[
            pltpu.VMEM((K, _CH), jnp.int32),
            pltpu.VMEM((K, _CH), jnp.int32),
            [pltpu.VMEM((_CH, 16), _F32) for _ in range(NBUF)],
            pltpu.VMEM((RPT, 16), _F32),
            pltpu.VMEM_SHARED((NACC, 16), _F32),
            pltpu.SemaphoreType.DMA,
            pltpu.SemaphoreType.DMA,
        ],
    )
    return kern(table, src2d, dst2d, zero)


def _restrict_body(h_ref, cid_ref, zero128_ref, zero16_ref, ones_ref,
                   sums_ref, cnts_ref,
                   cidv, hbuf, onesb, stage, stage16, acc, acc16, gsem, ssem):
    cid = lax.axis_index("c")
    sid = lax.axis_index("s")
    wid = cid * _NSUB + sid
    RPT = _NPC // _NSUB
    pltpu.sync_copy(cid_ref.at[pl.ds(wid * _KR, _KR)], cidv)
    pltpu.sync_copy(ones_ref, onesb)
    pltpu.sync_copy(zero128_ref.at[pl.ds(sid * RPT, RPT)], stage)
    pltpu.sync_copy(stage, acc.at[pl.ds(sid * RPT, RPT)])
    pltpu.sync_copy(zero16_ref.at[pl.ds(sid * RPT, RPT)], stage16)
    pltpu.sync_copy(stage16, acc16.at[pl.ds(sid * RPT, RPT)])
    plsc.subcore_barrier()

    gd = [pltpu.async_copy(
              h_ref.at[pl.ds(wid * (_KR * _RCH) + k * _RCH, _RCH)],
              hbuf[k], gsem)
          for k in range(_KR)]
    sd = []
    for k in range(_KR):
        gd[k].wait()
        sd.append(pltpu.async_copy(hbuf[k], acc.at[cidv.at[k]],
                                   ssem, add=True))
        sd.append(pltpu.async_copy(onesb, acc16.at[cidv.at[k]],
                                   ssem, add=True))
    for d in sd:
        d.wait()
    plsc.subcore_barrier()
    pltpu.sync_copy(acc.at[pl.ds(sid * RPT, RPT)], stage)
    pltpu.sync_copy(stage, sums_ref.at[cid, pl.ds(sid * RPT, RPT)])
    pltpu.sync_copy(acc16.at[pl.ds(sid * RPT, RPT)], stage16)
    pltpu.sync_copy(stage16, cnts_ref.at[cid, pl.ds(sid * RPT, RPT)])


def _restrict(h, cid2d):
    RPT = _NPC // _NSUB
    mesh = plsc.VectorSubcoreMesh(core_axis_name="c", subcore_axis_name="s")
    zero128 = jnp.zeros((_NPC, 128), _F32)
    zero16 = jnp.zeros((_NPC, 16), _F32)
    ones = jnp.ones((_RCH, 16), _F32)
    kern = pl.kernel(
        _restrict_body,
        out_type=(jax.ShapeDtypeStruct((_NCORES, _NPC, 128), _F32),
                  jax.ShapeDtypeStruct((_NCORES, _NPC, 16), _F32)),
        mesh=mesh,
        compiler_params=pltpu.CompilerParams(use_tc_tiling_on_sc=False),
        scratch_types=[
            pltpu.VMEM((_KR, _RCH), jnp.int32),
            [pltpu.VMEM((_RCH, 128), _F32) for _ in range(_KR)],
            pltpu.VMEM((_RCH, 16), _F32),
            pltpu.VMEM((RPT, 128), _F32),
            pltpu.VMEM((RPT, 16), _F32),
            pltpu.VMEM_SHARED((_NPC, 128), _F32),
            pltpu.VMEM_SHARED((_NPC, 16), _F32),
            pltpu.SemaphoreType.DMA,
            pltpu.SemaphoreType.DMA,
        ],
    )
    return kern(h, cid2d, zero128, zero16, ones)


def _prolong_add_body(d2_ref, hd_ref, cid_ref, out_ref, cidv, hbuf,
                      gsem, ssem):
    cid = lax.axis_index("c")
    sid = lax.axis_index("s")
    wid = cid * _NSUB + sid
    pltpu.sync_copy(cid_ref.at[pl.ds(wid * _KR, _KR)], cidv)

    ld = [pltpu.async_copy(
              d2_ref.at[pl.ds(wid * (_KR * _RCH) + k * _RCH, _RCH)],
              hbuf[k], gsem)
          for k in range(_KR)]
    gd = []
    for k in range(_KR):
        ld[k].wait()
        gd.append(pltpu.async_copy(hd_ref.at[cidv.at[k]], hbuf[k],
                                   gsem, add=True))
    sd = []
    for k in range(_KR):
        gd[k].wait()
        sd.append(pltpu.async_copy(
            hbuf[k],
            out_ref.at[pl.ds(wid * (_KR * _RCH) + k * _RCH, _RCH)],
            ssem))
    for d in sd:
        d.wait()


def _prolong_add(d2, hd, cid2d):
    mesh = plsc.VectorSubcoreMesh(core_axis_name="c", subcore_axis_name="s")
    kern = pl.kernel(
        _prolong_add_body,
        out_type=jax.ShapeDtypeStruct((_NPF, 128), _F32),
        mesh=mesh,
        compiler_params=pltpu.CompilerParams(use_tc_tiling_on_sc=False),
        scratch_types=[
            pltpu.VMEM((_KR, _RCH), jnp.int32),
            [pltpu.VMEM((_RCH, 128), _F32) for _ in range(_KR)],
            pltpu.SemaphoreType.DMA,
            pltpu.SemaphoreType.DMA,
        ],
    )
    return kern(d2, hd, cid2d)


def _embed_body(x_ref, wet_ref, be_ref, vt_ref, h_ref, g_ref):
    h = jnp.maximum(_dot(x_ref[...], wet_ref[...]) + be_ref[...], 0.0)
    h_ref[...] = h
    g_ref[...] = _dot(h, vt_ref[...])


def _tc_embed(x, Wet, be, Vt, BM):
    R = x.shape[0]
    grid = R // BM
    return pl.pallas_call(
        _embed_body,
        grid=(grid,),
        in_specs=[
            pl.BlockSpec((BM, _D), lambda i: (i, 0)),
            pl.BlockSpec((_D, _H), lambda i: (0, 0)),
            pl.BlockSpec((1, _H), lambda i: (0, 0)),
            pl.BlockSpec((_H, _BOND), lambda i: (0, 0)),
        ],
        out_specs=[
            pl.BlockSpec((BM, _H), lambda i: (i, 0)),
            pl.BlockSpec((BM, _BOND), lambda i: (i, 0)),
        ],
        out_shape=[
            jax.ShapeDtypeStruct((R, _H), _F32),
            jax.ShapeDtypeStruct((R, _BOND), _F32),
        ],
    )(x, Wet, be, Vt)


def _layer_body(h_ref, a0_ref, a1_ref, wt_ref, b_ref, ut_ref, extra_ref,
                gb_ref, hn_ref, g_ref):
    agg = a0_ref[0] + a1_ref[0]
    hn = _dot(h_ref[...], wt_ref[...]) + b_ref[...] + _dot(agg, ut_ref[...])
    hn = jnp.maximum(hn, 0.0)
    hn_ref[...] = hn
    if g_ref is not None:
        g_ref[...] = _dot(hn, extra_ref[...]) + gb_ref[...]


def _plain_layer_body(h_ref, a0_ref, a1_ref, wt_ref, b_ref, ut_ref,
                      hn_ref):
    _layer_body(h_ref, a0_ref, a1_ref, wt_ref, b_ref, ut_ref, None, None,
                hn_ref, None)


def _tc_layer(h, agg, Wt, b, Ut, extra, gb, BM, gdim):
    R = h.shape[0]
    grid = R // BM
    in_specs = [
        pl.BlockSpec((BM, _H), lambda i: (i, 0)),
        pl.BlockSpec((1, BM, 16), lambda i: (0, i, 0)),
        pl.BlockSpec((1, BM, 16), lambda i: (1, i, 0)),
        pl.BlockSpec((_H, _H), lambda i: (0, 0)),
        pl.BlockSpec((1, _H), lambda i: (0, 0)),
        pl.BlockSpec((_BOND, _H), lambda i: (0, 0)),
    ]
    args = [h, agg, agg, Wt, b, Ut]
    out_shape = [jax.ShapeDtypeStruct((R, _H), _F32)]
    out_specs = [pl.BlockSpec((BM, _H), lambda i: (i, 0))]
    if extra is not None:
        in_specs.append(pl.BlockSpec((_H, gdim), lambda i: (0, 0)))
        in_specs.append(pl.BlockSpec((1, gdim), lambda i: (0, 0)))
        args.extend([extra, gb])
        out_shape.append(jax.ShapeDtypeStruct((R, gdim), _F32))
        out_specs.append(pl.BlockSpec((BM, gdim), lambda i: (i, 0)))
        body = _layer_body
    else:
        body = _plain_layer_body
    res = pl.pallas_call(
        body,
        grid=(grid,),
        in_specs=in_specs,
        out_specs=out_specs,
        out_shape=out_shape,
    )(*args)
    return res if extra is not None else (res[0], None)


def _mean_body(s0_ref, s1_ref, c0_ref, c1_ref, vt_ref, hc_ref, g_ref):
    cnt = c0_ref[0][:, 0:1] + c1_ref[0][:, 0:1]
    hc = (s0_ref[0] + s1_ref[0]) / jnp.maximum(cnt, 1.0)
    hc_ref[...] = hc
    g_ref[...] = _dot(hc, vt_ref[...])


def _tc_mean(sums, cnts, Vt):
    R = sums.shape[1]
    return pl.pallas_call(
        _mean_body,
        grid=(1,),
        in_specs=[
            pl.BlockSpec((1, R, _H), lambda i: (0, 0, 0)),
            pl.BlockSpec((1, R, _H), lambda i: (1, 0, 0)),
            pl.BlockSpec((1, R, 16), lambda i: (0, 0, 0)),
            pl.BlockSpec((1, R, 16), lambda i: (1, 0, 0)),
            pl.BlockSpec((_H, _BOND), lambda i: (0, 0)),
        ],
        out_specs=[
            pl.BlockSpec((R, _H), lambda i: (0, 0)),
            pl.BlockSpec((R, _BOND), lambda i: (0, 0)),
        ],
        out_shape=[
            jax.ShapeDtypeStruct((R, _H), _F32),
            jax.ShapeDtypeStruct((R, _BOND), _F32),
        ],
    )(sums, sums, cnts, cnts, Vt)


def _pad_edges(src, dst, E, EP, nsrc, dummy, ndummy):
    pad = jnp.arange(EP - E, dtype=jnp.int32)
    src_p = jnp.concatenate([src, pad % nsrc])
    dst_p = jnp.concatenate([dst, dummy + pad % ndummy])
    return src_p.reshape(-1, _CH), dst_p.reshape(-1, _CH)


@jax.jit
def kernel(x_fine, edge_index_fine, edge_index_coarse, cluster_ids,
           W_embed, b_embed,
           Wf0, bf0, Vf0, Uf0, Wf1, bf1, Vf1, Uf1,
           Wc0, bc0, Vc0, Uc0, Wc1, bc1, Vc1, Uc1, W_dec, b_dec):
    x = jnp.zeros((_NPF, _D), _F32).at[:_NF].set(x_fine)
    srcf2d, dstf2d = _pad_edges(edge_index_fine[0], edge_index_fine[1],
                                _EF, _EFP, _NF, _NPF, _AF - _NPF)
    srcc2d, dstc2d = _pad_edges(edge_index_coarse[0], edge_index_coarse[1],
                                _EC, _ECP, _NC, _NC, _NPC - _NC)
    padc = jnp.arange(_NPF - _NF, dtype=jnp.int32)
    cidp = jnp.concatenate([cluster_ids, _NC + padc % (_NPC - _NC)])
    cid2d = cidp.reshape(_NW * _KR, _RCH)

    zb16 = jnp.zeros((1, _BOND), _F32)
    bd = b_dec.reshape(1, _D)

    BMF = 1280

    h0, g0 = _tc_embed(x, W_embed.T, b_embed.reshape(1, _H), Vf0.T, BMF)

    agg0 = _seg_sum16(g0, srcf2d, dstf2d, K=_KF, NACC=_AF)
    h1, g1 = _tc_layer(h0, agg0, Wf0.T, bf0.reshape(1, _H),
                       Uf0.T, Vf1.T, zb16, BMF, _BOND)

    agg1 = _seg_sum16(g1, srcf2d, dstf2d, K=_KF, NACC=_AF)
    h2, d2 = _tc_layer(h1, agg1, Wf1.T, bf1.reshape(1, _H),
                       Uf1.T, W_dec.T, bd, BMF, _D)

    sums, cnts = _restrict(h2, cid2d)
    hc0, gc0 = _tc_mean(sums, cnts, Vc0.T)

    aggc0 = _seg_sum16(gc0, srcc2d, dstc2d, K=_KC, NACC=_AC)
    hc1, gc1 = _tc_layer(hc0, aggc0, Wc0.T, bc0.reshape(1, _H),
                         Uc0.T, Vc1.T, zb16, _NPC, _BOND)

    aggc1 = _seg_sum16(gc1, srcc2d, dstc2d, K=_KC, NACC=_AC)
    _, hd = _tc_layer(hc1, aggc1, Wc1.T, bc1.reshape(1, _H), Uc1.T,
                      W_dec.T, jnp.zeros((1, _D), _F32), _NPC, _D)

    out = _prolong_add(d2, hd, cid2d)
    return out[:_NF]


# speedup vs baseline: 17.6412x; 1.1084x over previous
import functools

import jax
import jax.numpy as jnp
from jax import lax
from jax.experimental import pallas as pl
from jax.experimental.pallas import tpu as pltpu
from jax.experimental.pallas import tpu_sc as plsc

_NF = 10000
_NC = 2500
_EF = 320000
_EC = 40000
_D = 128
_H = 128
_BOND = 16

_NCORES = 2
_NSUB = 16
_NW = _NCORES * _NSUB

_NPC = 2560

_CH = 128
_KF = 80
_CHF = _NW * _KF
_KC = 10
_CHC = _NW * _KC
_AF = 10368
_AC = 2560

_RCH = 40
_KR = 8
_NCHR = _NF // _RCH

_F32 = jnp.float32
_ECP_PAD = _NW * _KC * _CH - _EC


def _dot_t(a, b):
    return lax.dot_general(a, b, (((1,), (1,)), ((), ())),
                           preferred_element_type=_F32)


def _seg_sum16_body(table_ref, src_ref, dst_ref, zero_ref, out_ref,
                    srcv, dstv, rows, stage, acc, gsem, ssem, *, K, RPT, NBUF):
    cid = lax.axis_index("c")
    sid = lax.axis_index("s")
    wid = cid * _NSUB + sid
    pltpu.sync_copy(src_ref.at[wid], srcv)
    pltpu.sync_copy(dst_ref.at[wid], dstv)
    pltpu.sync_copy(zero_ref.at[pl.ds(sid * RPT, RPT)], stage)
    pltpu.sync_copy(stage, acc.at[pl.ds(sid * RPT, RPT)])
    plsc.subcore_barrier()

    def group(g, carry):
        k0 = g * NBUF
        gd = [pltpu.async_copy(table_ref.at[srcv.at[k0 + b]], rows[b], gsem)
              for b in range(NBUF)]
        sd = []
        for b in range(NBUF):
            gd[b].wait()
            sd.append(pltpu.async_copy(rows[b], acc.at[dstv.at[k0 + b]],
                                       ssem, add=True))
        for b in range(NBUF):
            sd[b].wait()
        return carry

    lax.fori_loop(0, K // NBUF, group, 0)
    plsc.subcore_barrier()
    pltpu.sync_copy(acc.at[pl.ds(sid * RPT, RPT)], stage)
    pltpu.sync_copy(stage, out_ref.at[cid, pl.ds(sid * RPT, RPT)])


def _seg_sum16(table, src3, dst3, *, K, NACC, NBUF):
    RPT = NACC // _NSUB
    mesh = plsc.VectorSubcoreMesh(core_axis_name="c", subcore_axis_name="s")
    zero = jnp.zeros((NACC, 16), _F32)
    kern = pl.kernel(
        functools.partial(_seg_sum16_body, K=K, RPT=RPT, NBUF=NBUF),
        out_type=jax.ShapeDtypeStruct((_NCORES, NACC, 16), _F32),
        mesh=mesh,
        compiler_params=pltpu.CompilerParams(use_tc_tiling_on_sc=False),
        scratch_types=[
            pltpu.VMEM((K, _CH), jnp.int32),
            pltpu.VMEM((K, _CH), jnp.int32),
            [pltpu.VMEM((_CH, 16), _F32) for _ in range(NBUF)],
            pltpu.VMEM((RPT, 16), _F32),
            pltpu.VMEM_SHARED((NACC, 16), _F32),
            pltpu.SemaphoreType.DMA,
            pltpu.SemaphoreType.DMA,
        ],
    )
    return kern(table, src3, dst3, zero)


def _restrict_body(h_ref, cid_ref, zero128_ref, zero16_ref, ones_ref,
                   sums_ref, cnts_ref,
                   cidv, hbuf, onesb, stage, stage16, acc, acc16, gsem, ssem):
    cid = lax.axis_index("c")
    sid = lax.axis_index("s")
    wid = cid * _NSUB + sid
    RPT = _NPC // _NSUB
    pltpu.sync_copy(cid_ref.at[pl.ds(wid * _KR, _KR)], cidv)
    pltpu.sync_copy(ones_ref, onesb)
    pltpu.sync_copy(zero128_ref.at[pl.ds(sid * RPT, RPT)], stage)
    pltpu.sync_copy(stage, acc.at[pl.ds(sid * RPT, RPT)])
    pltpu.sync_copy(zero16_ref.at[pl.ds(sid * RPT, RPT)], stage16)
    pltpu.sync_copy(stage16, acc16.at[pl.ds(sid * RPT, RPT)])
    plsc.subcore_barrier()

    def h_src(k):
        return h_ref.at[pl.ds(wid * (_KR * _RCH) + k * _RCH, _RCH)]

    for k in range(_KR):
        @pl.when(wid * _KR + k < _NCHR)
        def _(k=k):
            pltpu.async_copy(h_src(k), hbuf[k], gsem)
    for k in range(_KR):
        @pl.when(wid * _KR + k < _NCHR)
        def _(k=k):
            pltpu.make_async_copy(h_src(k), hbuf[k], gsem).wait()
            pltpu.async_copy(hbuf[k], acc.at[cidv.at[k]], ssem, add=True)
            pltpu.async_copy(onesb, acc16.at[cidv.at[k]], ssem, add=True)
    for k in range(_KR):
        @pl.when(wid * _KR + k < _NCHR)
        def _(k=k):
            pltpu.make_async_copy(hbuf[k], acc.at[cidv.at[k]], ssem).wait()
            pltpu.make_async_copy(onesb, acc16.at[cidv.at[k]], ssem).wait()
    plsc.subcore_barrier()
    pltpu.sync_copy(acc.at[pl.ds(sid * RPT, RPT)], stage)
    pltpu.sync_copy(stage, sums_ref.at[cid, pl.ds(sid * RPT, RPT)])
    pltpu.sync_copy(acc16.at[pl.ds(sid * RPT, RPT)], stage16)
    pltpu.sync_copy(stage16, cnts_ref.at[cid, pl.ds(sid * RPT, RPT)])


def _restrict(h, cid2d):
    RPT = _NPC // _NSUB
    mesh = plsc.VectorSubcoreMesh(core_axis_name="c", subcore_axis_name="s")
    zero128 = jnp.zeros((_NPC, 128), _F32)
    zero16 = jnp.zeros((_NPC, 16), _F32)
    ones = jnp.ones((_RCH, 16), _F32)
    kern = pl.kernel(
        _restrict_body,
        out_type=(jax.ShapeDtypeStruct((_NCORES, _NPC, 128), _F32),
                  jax.ShapeDtypeStruct((_NCORES, _NPC, 16), _F32)),
        mesh=mesh,
        compiler_params=pltpu.CompilerParams(use_tc_tiling_on_sc=False),
        scratch_types=[
            pltpu.VMEM((_KR, _RCH), jnp.int32),
            [pltpu.VMEM((_RCH, 128), _F32) for _ in range(_KR)],
            pltpu.VMEM((_RCH, 16), _F32),
            pltpu.VMEM((RPT, 128), _F32),
            pltpu.VMEM((RPT, 16), _F32),
            pltpu.VMEM_SHARED((_NPC, 128), _F32),
            pltpu.VMEM_SHARED((_NPC, 16), _F32),
            pltpu.SemaphoreType.DMA,
            pltpu.SemaphoreType.DMA,
        ],
    )
    return kern(h, cid2d, zero128, zero16, ones)


def _prolong_add_body(d2_ref, hd_ref, cid_ref, out_ref, cidv, hbuf,
                      gsem, ssem):
    cid = lax.axis_index("c")
    sid = lax.axis_index("s")
    wid = cid * _NSUB + sid
    pltpu.sync_copy(cid_ref.at[pl.ds(wid * _KR, _KR)], cidv)

    def rows(k):
        return pl.ds(wid * (_KR * _RCH) + k * _RCH, _RCH)

    for k in range(_KR):
        @pl.when(wid * _KR + k < _NCHR)
        def _(k=k):
            pltpu.async_copy(d2_ref.at[rows(k)], hbuf[k], gsem)
    for k in range(_KR):
        @pl.when(wid * _KR + k < _NCHR)
        def _(k=k):
            pltpu.make_async_copy(d2_ref.at[rows(k)], hbuf[k], gsem).wait()
            pltpu.async_copy(hd_ref.at[cidv.at[k]], hbuf[k], gsem, add=True)
    for k in range(_KR):
        @pl.when(wid * _KR + k < _NCHR)
        def _(k=k):
            pltpu.make_async_copy(hd_ref.at[cidv.at[k]], hbuf[k], gsem).wait()
            pltpu.async_copy(hbuf[k], out_ref.at[rows(k)], ssem)
    for k in range(_KR):
        @pl.when(wid * _KR + k < _NCHR)
        def _(k=k):
            pltpu.make_async_copy(hbuf[k], out_ref.at[rows(k)], ssem).wait()


def _prolong_add(d2, hd, cid2d):
    mesh = plsc.VectorSubcoreMesh(core_axis_name="c", subcore_axis_name="s")
    kern = pl.kernel(
        _prolong_add_body,
        out_type=jax.ShapeDtypeStruct((_NF, 128), _F32),
        mesh=mesh,
        compiler_params=pltpu.CompilerParams(use_tc_tiling_on_sc=False),
        scratch_types=[
            pltpu.VMEM((_KR, _RCH), jnp.int32),
            [pltpu.VMEM((_RCH, 128), _F32) for _ in range(_KR)],
            pltpu.SemaphoreType.DMA,
            pltpu.SemaphoreType.DMA,
        ],
    )
    return kern(d2, hd, cid2d)


def _prep_body(ef_ref, cid_ref, srcf_ref, dstf_ref, cid2_ref):
    nps = _CHF - _EF // _CH
    g = (lax.broadcasted_iota(jnp.int32, (nps, _CH), 0) * _CH
         + lax.broadcasted_iota(jnp.int32, (nps, _CH), 1))
    srcf_ref[...] = jnp.concatenate([ef_ref[0], g % _NF], axis=0)
    dstf_ref[...] = jnp.concatenate([ef_ref[1], _NF + g % (_AF - _NF)],
                                    axis=0)
    g2 = (lax.broadcasted_iota(jnp.int32, (6, _RCH), 0) * _RCH
          + lax.broadcasted_iota(jnp.int32, (6, _RCH), 1))
    cid2_ref[...] = jnp.concatenate([cid_ref[...], _NC + g2 % (_NPC - _NC)],
                                    axis=0)


def _tc_prep(ef3, cidr):
    nef = _EF // _CH
    return pl.pallas_call(
        _prep_body,
        grid=(1,),
        in_specs=[
            pl.BlockSpec((2, nef, _CH), lambda i: (0, 0, 0)),
            pl.BlockSpec((_NCHR, _RCH), lambda i: (0, 0)),
        ],
        out_specs=[
            pl.BlockSpec((_CHF, _CH), lambda i: (0, 0)),
            pl.BlockSpec((_CHF, _CH), lambda i: (0, 0)),
            pl.BlockSpec((_NCHR + 6, _RCH), lambda i: (0, 0)),
        ],
        out_shape=[
            jax.ShapeDtypeStruct((_CHF, _CH), jnp.int32),
            jax.ShapeDtypeStruct((_CHF, _CH), jnp.int32),
            jax.ShapeDtypeStruct((_NCHR + 6, _RCH), jnp.int32),
        ],
    )(ef3, cidr)


def _embed_body(x_ref, we_ref, be_ref, v_ref, h_ref, g_ref):
    h = jnp.maximum(_dot_t(x_ref[...], we_ref[...]) + be_ref[...], 0.0)
    h_ref[...] = h
    g_ref[...] = _dot_t(h, v_ref[...])


def _tc_embed(x, We, be, V, BM):
    R = x.shape[0]
    grid = R // BM
    return pl.pallas_call(
        _embed_body,
        grid=(grid,),
        in_specs=[
            pl.BlockSpec((BM, _D), lambda i: (i, 0)),
            pl.BlockSpec((_H, _D), lambda i: (0, 0)),
            pl.BlockSpec((1, _H), lambda i: (0, 0)),
            pl.BlockSpec((_BOND, _H), lambda i: (0, 0)),
        ],
        out_specs=[
            pl.BlockSpec((BM, _H), lambda i: (i, 0)),
            pl.BlockSpec((BM, _BOND), lambda i: (i, 0)),
        ],
        out_shape=[
            jax.ShapeDtypeStruct((R, _H), _F32),
            jax.ShapeDtypeStruct((R, _BOND), _F32),
        ],
    )(x, We, be, V)


def _layer_body(h_ref, a0_ref, a1_ref, w_ref, b_ref, u_ref, extra_ref,
                gb_ref, hn_ref, g_ref):
    agg = a0_ref[0] + a1_ref[0]
    hn = _dot_t(h_ref[...], w_ref[...]) + b_ref[...] + _dot_t(agg, u_ref[...])
    hn = jnp.maximum(hn, 0.0)
    hn_ref[...] = hn
    if g_ref is not None:
        g_ref[...] = _dot_t(hn, extra_ref[...]) + gb_ref[...]


def _tc_layer(h, agg, W, b, U, extra, gb, BM, gdim):
    R = h.shape[0]
    grid = R // BM
    in_specs = [
        pl.BlockSpec((BM, _H), lambda i: (i, 0)),
        pl.BlockSpec((1, BM, 16), lambda i: (0, i, 0)),
        pl.BlockSpec((1, BM, 16), lambda i: (1, i, 0)),
        pl.BlockSpec((_H, _H), lambda i: (0, 0)),
        pl.BlockSpec((1, _H), lambda i: (0, 0)),
        pl.BlockSpec((_H, _BOND), lambda i: (0, 0)),
        pl.BlockSpec((gdim, _H), lambda i: (0, 0)),
        pl.BlockSpec((1, gdim), lambda i: (0, 0)),
    ]
    args = [h, agg, agg, W, b, U, extra, gb]
    out_shape = [jax.ShapeDtypeStruct((R, _H), _F32),
                 jax.ShapeDtypeStruct((R, gdim), _F32)]
    out_specs = [pl.BlockSpec((BM, _H), lambda i: (i, 0)),
                 pl.BlockSpec((BM, gdim), lambda i: (i, 0))]
    return pl.pallas_call(
        _layer_body,
        grid=(grid,),
        in_specs=in_specs,
        out_specs=out_specs,
        out_shape=out_shape,
    )(*args)


def _mean_body(s0_ref, s1_ref, c0_ref, c1_ref, vt_ref, hc_ref, g_ref):
    cnt = c0_ref[0][:, 0:1] + c1_ref[0][:, 0:1]
    hc = (s0_ref[0] + s1_ref[0]) / jnp.maximum(cnt, 1.0)
    hc_ref[...] = hc
    g_ref[...] = _dot_t(hc, vt_ref[...])


def _tc_mean(sums, cnts, V):
    R = sums.shape[1]
    return pl.pallas_call(
        _mean_body,
        grid=(1,),
        in_specs=[
            pl.BlockSpec((1, R, _H), lambda i: (0, 0, 0)),
            pl.BlockSpec((1, R, _H), lambda i: (1, 0, 0)),
            pl.BlockSpec((1, R, 16), lambda i: (0, 0, 0)),
            pl.BlockSpec((1, R, 16), lambda i: (1, 0, 0)),
            pl.BlockSpec((_BOND, _H), lambda i: (0, 0)),
        ],
        out_specs=[
            pl.BlockSpec((R, _H), lambda i: (0, 0)),
            pl.BlockSpec((R, _BOND), lambda i: (0, 0)),
        ],
        out_shape=[
            jax.ShapeDtypeStruct((R, _H), _F32),
            jax.ShapeDtypeStruct((R, _BOND), _F32),
        ],
    )(sums, sums, cnts, cnts, V)


@jax.jit
def kernel(x_fine, edge_index_fine, edge_index_coarse, cluster_ids,
           W_embed, b_embed,
           Wf0, bf0, Vf0, Uf0, Wf1, bf1, Vf1, Uf1,
           Wc0, bc0, Vc0, Uc0, Wc1, bc1, Vc1, Uc1, W_dec, b_dec):
    ef3 = edge_index_fine.reshape(2, _EF // _CH, _CH)
    cidr = cluster_ids.reshape(_NCHR, _RCH)
    srcf, dstf, cid2d = _tc_prep(ef3, cidr)
    srcf3 = srcf.reshape(_NW, _KF, _CH)
    dstf3 = dstf.reshape(_NW, _KF, _CH)

    padc = jnp.arange(_ECP_PAD, dtype=jnp.int32)
    srcc3 = jnp.concatenate([edge_index_coarse[0], padc % _NC]) \
        .reshape(_NW, _KC, _CH)
    dstc3 = jnp.concatenate([edge_index_coarse[1],
                             _NC + padc % (_AC - _NC)]) \
        .reshape(_NW, _KC, _CH)

    zb16 = jnp.zeros((1, _BOND), _F32)
    bd = b_dec.reshape(1, _D)
    BMF = 2000

    h0, g0 = _tc_embed(x_fine, W_embed, b_embed.reshape(1, _H), Vf0, BMF)

    agg0 = _seg_sum16(g0, srcf3, dstf3, K=_KF, NACC=_AF, NBUF=8)
    h1, g1 = _tc_layer(h0, agg0, Wf0, bf0.reshape(1, _H),
                       Uf0, Vf1, zb16, BMF, _BOND)

    agg1 = _seg_sum16(g1, srcf3, dstf3, K=_KF, NACC=_AF, NBUF=8)
    h2, d2 = _tc_layer(h1, agg1, Wf1, bf1.reshape(1, _H),
                       Uf1, W_dec, bd, BMF, _D)

    sums, cnts = _restrict(h2, cid2d)
    hc0, gc0 = _tc_mean(sums, cnts, Vc0)

    aggc0 = _seg_sum16(gc0, srcc3, dstc3, K=_KC, NACC=_AC, NBUF=10)
    hc1, gc1 = _tc_layer(hc0, aggc0, Wc0, bc0.reshape(1, _H),
                         Uc0, Vc1, zb16, _NPC, _BOND)

    aggc1 = _seg_sum16(gc1, srcc3, dstc3, K=_KC, NACC=_AC, NBUF=10)
    _, hd = _tc_layer(hc1, aggc1, Wc1, bc1.reshape(1, _H), Uc1,
                      W_dec, jnp.zeros((1, _D), _F32), _NPC, _D)

    return _prolong_add(d2, hd, cid2d)
